# Initial kernel scaffold; baseline (speedup 1.0000x reference)
import functools

import jax
import jax.numpy as jnp
from jax import lax
from jax.experimental import pallas as pl
from jax.experimental.pallas import tpu as pltpu
from jax.experimental.pallas import tpu_sc as plsc

N_CLS = 1024
CTX_LEN = 77
DIM = 768
N_CTX = 20
SUFFIX_START = 1 + N_CTX
N_SUFFIX = CTX_LEN - SUFFIX_START
N_GATHER = 1 + N_SUFFIX
IDX_PAD = 64


@functools.lru_cache(maxsize=1)
def _build_sc_kernel():
    info = plsc.get_sparse_core_info()
    nw = info.num_cores * info.num_subcores
    cpw = N_CLS // nw
    mesh = plsc.VectorSubcoreMesh(core_axis_name="c", subcore_axis_name="s")

    @functools.partial(
        pl.kernel,
        mesh=mesh,
        out_type=jax.ShapeDtypeStruct((N_CLS, CTX_LEN, DIM), jnp.float32),
        scratch_types=[
            pltpu.VMEM((IDX_PAD,), jnp.int32),
            pltpu.VMEM((IDX_PAD, DIM), jnp.float32),
            pltpu.VMEM((N_CTX, DIM), jnp.float32),
            pltpu.SemaphoreType.DMA,
        ],
    )
    def body(idx_hbm, ctx_hbm, table_hbm, out_hbm, idx_v, rows_v, ctx_v, sem):
        wid = lax.axis_index("s") * info.num_cores + lax.axis_index("c")
        base = wid * cpw
        pltpu.sync_copy(ctx_hbm, ctx_v)

        def one_class(i, carry):
            n = base + i
            pltpu.sync_copy(idx_hbm.at[n], idx_v)
            pltpu.async_copy(table_hbm.at[idx_v], rows_v, sem).wait()
            pltpu.sync_copy(rows_v.at[pl.ds(0, 1)], out_hbm.at[n, pl.ds(0, 1)])
            pltpu.sync_copy(rows_v.at[pl.ds(1, N_SUFFIX)],
                            out_hbm.at[n, pl.ds(SUFFIX_START, N_SUFFIX)])
            pltpu.sync_copy(ctx_v, out_hbm.at[n, pl.ds(1, N_CTX)])
            return carry

        lax.fori_loop(0, cpw, one_class, 0)

    return body


def kernel(tokenized_prompts, token_embedding, ctx, ctx0, ctx1, mode):
    ctxs = jnp.where(mode == 0, ctx0, ctx1)
    ctx_full = jnp.concatenate([ctx, ctxs], axis=0)
    idx = jnp.concatenate(
        [tokenized_prompts[:, :1],
         tokenized_prompts[:, SUFFIX_START:],
         tokenized_prompts[:, :IDX_PAD - N_GATHER]], axis=1)
    prompts = _build_sc_kernel()(idx, ctx_full, token_embedding)
    return (prompts, tokenized_prompts)


# speedup vs baseline: 1.4025x; 1.4025x over previous
import functools

import jax
import jax.numpy as jnp
from jax import lax
from jax.experimental import pallas as pl
from jax.experimental.pallas import tpu as pltpu
from jax.experimental.pallas import tpu_sc as plsc

N_CLS = 1024
CTX_LEN = 77
DIM = 768
N_CTX = 20
SUFFIX_START = 1 + N_CTX
N_SUFFIX = CTX_LEN - SUFFIX_START
N_GATHER = 1 + N_SUFFIX
IDX_PAD = 64


@functools.lru_cache(maxsize=1)
def _build_sc_kernel():
    info = plsc.get_sparse_core_info()
    nw = info.num_cores * info.num_subcores
    cpw = N_CLS // nw
    mesh = plsc.VectorSubcoreMesh(core_axis_name="c", subcore_axis_name="s")

    @functools.partial(
        pl.kernel,
        mesh=mesh,
        out_type=jax.ShapeDtypeStruct((N_CLS, CTX_LEN, DIM), jnp.float32),
        compiler_params=pltpu.CompilerParams(use_tc_tiling_on_sc=False),
        scratch_types=[
            pltpu.VMEM((IDX_PAD,), jnp.int32),
            pltpu.VMEM((IDX_PAD, DIM), jnp.float32),
            pltpu.VMEM((N_CTX, DIM), jnp.float32),
            pltpu.SemaphoreType.DMA,
        ],
    )
    def body(idx_hbm, ctx_hbm, table_hbm, out_hbm, idx_v, rows_v, ctx_v, sem):
        wid = lax.axis_index("s") * info.num_cores + lax.axis_index("c")
        base = wid * cpw
        pltpu.sync_copy(ctx_hbm, ctx_v)

        def one_class(i, carry):
            n = base + i
            pltpu.sync_copy(idx_hbm.at[n], idx_v)
            pltpu.async_copy(table_hbm.at[idx_v], rows_v, sem).wait()
            pltpu.sync_copy(rows_v.at[pl.ds(0, 1)], out_hbm.at[n, pl.ds(0, 1)])
            pltpu.sync_copy(rows_v.at[pl.ds(1, N_SUFFIX)],
                            out_hbm.at[n, pl.ds(SUFFIX_START, N_SUFFIX)])
            pltpu.sync_copy(ctx_v, out_hbm.at[n, pl.ds(1, N_CTX)])
            return carry

        lax.fori_loop(0, cpw, one_class, 0)

    return body


def kernel(tokenized_prompts, token_embedding, ctx, ctx0, ctx1, mode):
    ctxs = jnp.where(mode == 0, ctx0, ctx1)
    ctx_full = jnp.concatenate([ctx, ctxs], axis=0)
    idx = jnp.concatenate(
        [tokenized_prompts[:, :1],
         tokenized_prompts[:, SUFFIX_START:],
         tokenized_prompts[:, :IDX_PAD - N_GATHER]], axis=1)
    prompts = _build_sc_kernel()(idx, ctx_full, token_embedding)
    return (prompts, tokenized_prompts)


# speedup vs baseline: 1.4690x; 1.0474x over previous
import functools

import jax
import jax.numpy as jnp
from jax import lax
from jax.experimental import pallas as pl
from jax.experimental.pallas import tpu as pltpu
from jax.experimental.pallas import tpu_sc as plsc

N_CLS = 1024
CTX_LEN = 77
DIM = 768
N_CTX = 20
SUFFIX_START = 1 + N_CTX
N_SUFFIX = CTX_LEN - SUFFIX_START
IDX_PAD = 64
SUF_OFF = 8


@functools.lru_cache(maxsize=1)
def _build_sc_kernel():
    info = plsc.get_sparse_core_info()
    nw = info.num_cores * info.num_subcores
    cpw = N_CLS // nw
    mesh = plsc.VectorSubcoreMesh(core_axis_name="c", subcore_axis_name="s")

    @functools.partial(
        pl.kernel,
        mesh=mesh,
        out_type=jax.ShapeDtypeStruct((N_CLS, CTX_LEN, DIM), jnp.float32),
        compiler_params=pltpu.CompilerParams(use_tc_tiling_on_sc=False),
        scratch_types=[
            pltpu.VMEM((cpw, IDX_PAD), jnp.int32),
            pltpu.VMEM((CTX_LEN, DIM), jnp.float32),
            pltpu.VMEM((CTX_LEN, DIM), jnp.float32),
            pltpu.SemaphoreType.DMA,
            pltpu.SemaphoreType.DMA,
            pltpu.SemaphoreType.DMA,
        ],
    )
    def body(idx_hbm, ctx_hbm, table_hbm, out_hbm,
             idx_v, buf0, buf1, gsem, osem0, osem1):
        wid = lax.axis_index("s") * info.num_cores + lax.axis_index("c")
        base = wid * cpw
        bufs = (buf0, buf1)
        osems = (osem0, osem1)

        pltpu.sync_copy(idx_hbm.at[pl.ds(base, cpw)], idx_v)
        pltpu.sync_copy(ctx_hbm, buf0.at[pl.ds(1, N_CTX)])
        pltpu.sync_copy(ctx_hbm, buf1.at[pl.ds(1, N_CTX)])

        def gather_and_store(i, b):
            n = base + i
            buf = bufs[b]
            g0 = pltpu.async_copy(
                table_hbm.at[idx_v.at[i, pl.ds(0, 1)]],
                buf.at[pl.ds(0, 1)], gsem)
            g1 = pltpu.async_copy(
                table_hbm.at[idx_v.at[i, pl.ds(SUF_OFF, N_SUFFIX)]],
                buf.at[pl.ds(SUFFIX_START, N_SUFFIX)], gsem)
            g0.wait()
            g1.wait()
            pltpu.async_copy(buf, out_hbm.at[n], osems[b])

        gather_and_store(0, 0)
        gather_and_store(1, 1)

        @pl.loop(1, cpw // 2)
        def _(p):
            for b in range(2):
                i = 2 * p + b
                pltpu.make_async_copy(bufs[b], out_hbm.at[base + i],
                                      osems[b]).wait()
                gather_and_store(i, b)

        pltpu.make_async_copy(buf0, out_hbm.at[base], osem0).wait()
        pltpu.make_async_copy(buf1, out_hbm.at[base], osem1).wait()

    return body


def kernel(tokenized_prompts, token_embedding, ctx, ctx0, ctx1, mode):
    ctxs = jnp.where(mode == 0, ctx0, ctx1)
    ctx_full = jnp.concatenate([ctx, ctxs], axis=0)
    idx = jnp.concatenate(
        [tokenized_prompts[:, :1],
         tokenized_prompts[:, 1:SUF_OFF],
         tokenized_prompts[:, SUFFIX_START:]], axis=1)
    prompts = _build_sc_kernel()(idx, ctx_full, token_embedding)
    return (prompts, tokenized_prompts)


# speedup vs baseline: 2.8931x; 1.9694x over previous
import functools

import jax
import jax.numpy as jnp
from jax import lax
from jax.experimental import pallas as pl
from jax.experimental.pallas import tpu as pltpu
from jax.experimental.pallas import tpu_sc as plsc

N_CLS = 1024
CTX_LEN = 77
DIM = 768
N_CTX = 20
SUFFIX_START = 1 + N_CTX
MAIN_LO = 24
MAIN_N = 48
MSTRIDE = 48
HSTRIDE = 16
LANES = 16
PATCH8 = (0, 21, 22, 23, 72, 73, 74, 75)


@functools.lru_cache(maxsize=1)
def _build_sc_kernel():
    info = plsc.get_sparse_core_info()
    nw = info.num_cores * info.num_subcores
    cpw = N_CLS // nw
    mesh = plsc.VectorSubcoreMesh(core_axis_name="c", subcore_axis_name="s")

    @functools.partial(
        pl.kernel,
        mesh=mesh,
        out_type=jax.ShapeDtypeStruct((N_CLS, CTX_LEN, DIM), jnp.float32),
        scratch_types=[
            pltpu.VMEM((cpw // 2 * HSTRIDE,), jnp.int32),
            pltpu.VMEM((cpw // 2 * MSTRIDE,), jnp.int32),
            pltpu.VMEM((CTX_LEN, DIM), jnp.float32),
            pltpu.VMEM((CTX_LEN, DIM), jnp.float32),
            pltpu.VMEM((8, DIM), jnp.float32),
            pltpu.VMEM((1, DIM), jnp.float32),
            pltpu.SemaphoreType.DMA,
            pltpu.SemaphoreType.DMA,
            pltpu.SemaphoreType.DMA,
        ],
    )
    def body(idxh_hbm, idxm_hbm, ctx_hbm, table_hbm, out_hbm,
             idxh_v, idxm_v, buf0, buf1, sc8, sc1,
             gsem, osem0, osem1):
        wid = lax.axis_index("s") * info.num_cores + lax.axis_index("c")
        base = wid * cpw
        bufs = (buf0, buf1)
        osems = (osem0, osem1)

        hh = cpw // 2 * HSTRIDE
        mh = cpw // 2 * MSTRIDE

        def refill(half):
            pltpu.sync_copy(
                idxh_hbm.at[pl.ds(wid * (cpw * HSTRIDE) + half * hh, hh)],
                idxh_v)
            pltpu.sync_copy(
                idxm_hbm.at[pl.ds(wid * (cpw * MSTRIDE) + half * mh, mh)],
                idxm_v)

        refill(0)
        pltpu.sync_copy(ctx_hbm, buf0.at[pl.ds(0, 24)])
        pltpu.sync_copy(ctx_hbm, buf1.at[pl.ds(0, 24)])

        def do_class(i, b, drain):
            n = base + i
            j = lax.rem(i, cpw // 2)
            if drain:
                pltpu.make_async_copy(bufs[b], out_hbm.at[n], osems[b]).wait()
            g1 = pltpu.async_copy(
                table_hbm.at[idxh_v.at[pl.ds(pl.multiple_of(j * HSTRIDE, 8), 8)]],
                sc8, gsem)
            g2 = pltpu.async_copy(
                table_hbm.at[idxh_v.at[pl.ds(pl.multiple_of(j * HSTRIDE, 8) + 8,
                                             1)]],
                sc1, gsem)
            g3 = pltpu.async_copy(
                table_hbm.at[idxm_v.at[pl.ds(pl.multiple_of(j * MSTRIDE, 8),
                                             MAIN_N)]],
                bufs[b].at[pl.ds(MAIN_LO, MAIN_N)], gsem)
            g1.wait()
            g2.wait()
            g3.wait()
            for r, dst in enumerate(PATCH8):
                for c in range(DIM // LANES):
                    bufs[b][dst, pl.ds(LANES * c, LANES)] = \
                        sc8[r, pl.ds(LANES * c, LANES)]
            for c in range(DIM // LANES):
                bufs[b][CTX_LEN - 1, pl.ds(LANES * c, LANES)] = \
                    sc1[0, pl.ds(LANES * c, LANES)]
            pltpu.async_copy(bufs[b], out_hbm.at[n], osems[b])

        do_class(0, 0, False)
        do_class(1, 1, False)

        @pl.loop(1, cpw // 2)
        def _(p):
            @pl.when(p == cpw // 4)
            def _():
                refill(1)
            do_class(2 * p, 0, True)
            do_class(2 * p + 1, 1, True)

        pltpu.make_async_copy(buf0, out_hbm.at[base], osem0).wait()
        pltpu.make_async_copy(buf1, out_hbm.at[base], osem1).wait()

    return body


def kernel(tokenized_prompts, token_embedding, ctx, ctx0, ctx1, mode):
    tok = tokenized_prompts
    ctxs = jnp.where(mode == 0, ctx0, ctx1)
    idx_misc = jnp.concatenate(
        [tok[:, :1], tok[:, SUFFIX_START:MAIN_LO], tok[:, 72:],
         jnp.zeros((N_CLS, 7), jnp.int32)], axis=1).reshape(-1)
    idx_main = tok[:, MAIN_LO:MAIN_LO + MAIN_N].reshape(-1)
    zr = jnp.zeros((1, DIM), jnp.float32)
    ctx_t = jnp.concatenate([zr, ctx, ctxs, zr, zr, zr], axis=0)
    prompts = _build_sc_kernel()(idx_misc, idx_main, ctx_t, token_embedding)
    return (prompts, tokenized_prompts)
